# Initial kernel scaffold; baseline (speedup 1.0000x reference)
import functools

import jax
import jax.numpy as jnp
from jax import lax
from jax.experimental import pallas as pl
from jax.experimental.pallas import tpu as pltpu
from jax.experimental.pallas import tpu_sc as plsc

B = 4096
L_T = 50
L_D = 200
D_EMB = 64
D_BR = 16
HIDDEN = 128
OUT = 64

NC = 2
NS = 16
NW = NC * NS
IPW = B // NW


def _row_sum(rows_ref, n_rows):
    def body(r, accs):
        return tuple(accs[j] + rows_ref[r, pl.ds(j * 16, 16)] for j in range(4))
    init = tuple(jnp.zeros((16,), jnp.float32) for _ in range(4))
    return lax.fori_loop(0, n_rows, body, init)


def _sc_embed(id_idx, br_idx, t_idx, d_idx, id_tab, br_tab, t_tab, d_tab):
    mesh = plsc.VectorSubcoreMesh(core_axis_name="c", subcore_axis_name="s")

    @functools.partial(
        pl.kernel,
        out_type=(
            jax.ShapeDtypeStruct((B, D_EMB), jnp.float32),
            jax.ShapeDtypeStruct((B, D_EMB), jnp.float32),
            jax.ShapeDtypeStruct((B, D_EMB), jnp.float32),
            jax.ShapeDtypeStruct((B, D_BR), jnp.float32),
        ),
        mesh=mesh,
        scratch_types=[
            pltpu.VMEM((IPW,), jnp.int32),
            pltpu.VMEM((IPW,), jnp.int32),
            pltpu.VMEM((IPW, L_T), jnp.int32),
            pltpu.VMEM((IPW, L_D), jnp.int32),
            pltpu.VMEM((IPW, D_EMB), jnp.float32),
            pltpu.VMEM((IPW, D_BR), jnp.float32),
            pltpu.VMEM((L_T, D_EMB), jnp.float32),
            pltpu.VMEM((L_D, D_EMB), jnp.float32),
            pltpu.VMEM((IPW, D_EMB), jnp.float32),
            pltpu.VMEM((IPW, D_EMB), jnp.float32),
        ],
    )
    def k(id_idx_hbm, br_idx_hbm, t_idx_hbm, d_idx_hbm,
          id_tab_hbm, br_tab_hbm, t_tab_hbm, d_tab_hbm,
          id_out, t_out, d_out, br_out,
          idv, brv, tiv, div, idrows, brrows, trows, drows, tacc, dacc):
        wid = lax.axis_index("s") * NC + lax.axis_index("c")
        base = wid * IPW

        pltpu.sync_copy(id_idx_hbm.at[pl.ds(base, IPW)], idv)
        pltpu.sync_copy(id_tab_hbm.at[idv], idrows)
        pltpu.sync_copy(idrows, id_out.at[pl.ds(base, IPW)])

        pltpu.sync_copy(br_idx_hbm.at[pl.ds(base, IPW)], brv)
        pltpu.sync_copy(br_tab_hbm.at[brv], brrows)
        pltpu.sync_copy(brrows, br_out.at[pl.ds(base, IPW)])

        pltpu.sync_copy(t_idx_hbm.at[pl.ds(base, IPW)], tiv)
        pltpu.sync_copy(d_idx_hbm.at[pl.ds(base, IPW)], div)

        @pl.loop(0, IPW)
        def _(b):
            pltpu.sync_copy(t_tab_hbm.at[tiv.at[b]], trows)
            accs = _row_sum(trows, L_T)
            for j in range(4):
                tacc[b, pl.ds(j * 16, 16)] = accs[j]

        @pl.loop(0, IPW)
        def _(b):
            pltpu.sync_copy(d_tab_hbm.at[div.at[b, pl.ds(0, 128)]],
                            drows.at[pl.ds(0, 128)])
            pltpu.sync_copy(d_tab_hbm.at[div.at[b, pl.ds(128, L_D - 128)]],
                            drows.at[pl.ds(128, L_D - 128)])
            accs = _row_sum(drows, L_D)
            for j in range(4):
                dacc[b, pl.ds(j * 16, 16)] = accs[j]

        pltpu.sync_copy(tacc, t_out.at[pl.ds(base, IPW)])
        pltpu.sync_copy(dacc, d_out.at[pl.ds(base, IPW)])

    return k(id_idx, br_idx, t_idx, d_idx, id_tab, br_tab, t_tab, d_tab)


def _mlp_body(idr, tr, dr, brr, w1r, b1r, w2r, b2r, outr):
    w1 = w1r[...]
    h = jnp.dot(idr[...], w1[0:64], preferred_element_type=jnp.float32)
    h += (1.0 / L_T) * jnp.dot(tr[...], w1[64:128],
                               preferred_element_type=jnp.float32)
    h += (1.0 / L_D) * jnp.dot(dr[...], w1[128:192],
                               preferred_element_type=jnp.float32)
    h += jnp.dot(brr[...], w1[192:208], preferred_element_type=jnp.float32)
    h = jnp.maximum(h + b1r[...], 0.0)
    outr[...] = jnp.dot(h, w2r[...], preferred_element_type=jnp.float32) + b2r[...]


def _tc_mlp(id_emb, t_sum, d_sum, br_emb, W1, b1, W2, b2):
    blk = 512
    return pl.pallas_call(
        _mlp_body,
        grid=(B // blk,),
        in_specs=[
            pl.BlockSpec((blk, D_EMB), lambda i: (i, 0)),
            pl.BlockSpec((blk, D_EMB), lambda i: (i, 0)),
            pl.BlockSpec((blk, D_EMB), lambda i: (i, 0)),
            pl.BlockSpec((blk, D_BR), lambda i: (i, 0)),
            pl.BlockSpec((208, HIDDEN), lambda i: (0, 0)),
            pl.BlockSpec((1, HIDDEN), lambda i: (0, 0)),
            pl.BlockSpec((HIDDEN, OUT), lambda i: (0, 0)),
            pl.BlockSpec((1, OUT), lambda i: (0, 0)),
        ],
        out_specs=pl.BlockSpec((blk, OUT), lambda i: (i, 0)),
        out_shape=jax.ShapeDtypeStruct((B, OUT), jnp.float32),
    )(id_emb, t_sum, d_sum, br_emb, W1, b1.reshape(1, HIDDEN), W2,
      b2.reshape(1, OUT))


def kernel(product_id, product_title, product_description, product_brand,
           id_table, title_table, desc_table, brand_table, W1, b1, W2, b2):
    id_idx = product_id.astype(jnp.int32)
    br_idx = product_brand.astype(jnp.int32)
    t_idx = product_title.astype(jnp.int32)
    d_idx = product_description.astype(jnp.int32)
    id_emb, t_sum, d_sum, br_emb = _sc_embed(
        id_idx, br_idx, t_idx, d_idx,
        id_table, brand_table, title_table, desc_table)
    return _tc_mlp(id_emb, t_sum, d_sum, br_emb, W1, b1, W2, b2)


# speedup vs baseline: 3.6164x; 3.6164x over previous
import functools

import jax
import jax.numpy as jnp
from jax import lax
from jax.experimental import pallas as pl
from jax.experimental.pallas import tpu as pltpu
from jax.experimental.pallas import tpu_sc as plsc

B = 4096
L_T = 50
L_D = 200
D_EMB = 64
D_BR = 16
HIDDEN = 128
OUT = 64

NC = 2
NS = 16
NW = NC * NS
IPW = B // NW


def _row_sum(rows_ref, n_rows):
    def body(r, accs):
        return tuple(accs[j] + rows_ref[r, pl.ds(j * 16, 16)] for j in range(4))
    init = tuple(jnp.zeros((16,), jnp.float32) for _ in range(4))
    return lax.fori_loop(0, n_rows, body, init)


def _sc_embed(id_idx, br_idx, t_idx, d_idx, id_tab, br_tab, t_tab, d_tab):
    mesh = plsc.VectorSubcoreMesh(core_axis_name="c", subcore_axis_name="s")

    @functools.partial(
        pl.kernel,
        compiler_params=pltpu.CompilerParams(use_tc_tiling_on_sc=False),
        out_type=(
            jax.ShapeDtypeStruct((B, D_EMB), jnp.float32),
            jax.ShapeDtypeStruct((B, D_EMB), jnp.float32),
            jax.ShapeDtypeStruct((B, D_EMB), jnp.float32),
            jax.ShapeDtypeStruct((B, D_BR), jnp.float32),
        ),
        mesh=mesh,
        scratch_types=[
            pltpu.VMEM((IPW,), jnp.int32),
            pltpu.VMEM((IPW,), jnp.int32),
            pltpu.VMEM((IPW, L_T), jnp.int32),
            pltpu.VMEM((IPW, L_D), jnp.int32),
            pltpu.VMEM((IPW, D_EMB), jnp.float32),
            pltpu.VMEM((IPW, D_BR), jnp.float32),
            pltpu.VMEM((L_T, D_EMB), jnp.float32),
            pltpu.VMEM((L_D, D_EMB), jnp.float32),
            pltpu.VMEM((IPW, D_EMB), jnp.float32),
            pltpu.VMEM((IPW, D_EMB), jnp.float32),
        ],
    )
    def k(id_idx_hbm, br_idx_hbm, t_idx_hbm, d_idx_hbm,
          id_tab_hbm, br_tab_hbm, t_tab_hbm, d_tab_hbm,
          id_out, t_out, d_out, br_out,
          idv, brv, tiv, div, idrows, brrows, trows, drows, tacc, dacc):
        wid = lax.axis_index("s") * NC + lax.axis_index("c")
        base = wid * IPW

        pltpu.sync_copy(id_idx_hbm.at[pl.ds(base, IPW)], idv)
        pltpu.sync_copy(id_tab_hbm.at[idv], idrows)
        pltpu.sync_copy(idrows, id_out.at[pl.ds(base, IPW)])

        pltpu.sync_copy(br_idx_hbm.at[pl.ds(base, IPW)], brv)
        pltpu.sync_copy(br_tab_hbm.at[brv], brrows)
        pltpu.sync_copy(brrows, br_out.at[pl.ds(base, IPW)])

        pltpu.sync_copy(t_idx_hbm.at[pl.ds(base, IPW)], tiv)
        pltpu.sync_copy(d_idx_hbm.at[pl.ds(base, IPW)], div)

        @pl.loop(0, IPW)
        def _(b):
            pltpu.sync_copy(t_tab_hbm.at[tiv.at[b]], trows)
            accs = _row_sum(trows, L_T)
            for j in range(4):
                tacc[b, pl.ds(j * 16, 16)] = accs[j]

        @pl.loop(0, IPW)
        def _(b):
            pltpu.sync_copy(d_tab_hbm.at[div.at[b, pl.ds(0, 128)]],
                            drows.at[pl.ds(0, 128)])
            pltpu.sync_copy(d_tab_hbm.at[div.at[b, pl.ds(128, L_D - 128)]],
                            drows.at[pl.ds(128, L_D - 128)])
            accs = _row_sum(drows, L_D)
            for j in range(4):
                dacc[b, pl.ds(j * 16, 16)] = accs[j]

        pltpu.sync_copy(tacc, t_out.at[pl.ds(base, IPW)])
        pltpu.sync_copy(dacc, d_out.at[pl.ds(base, IPW)])

    return k(id_idx, br_idx, t_idx, d_idx, id_tab, br_tab, t_tab, d_tab)


def _mlp_body(idr, tr, dr, brr, w1r, b1r, w2r, b2r, outr):
    w1 = w1r[...]
    h = jnp.dot(idr[...], w1[0:64], preferred_element_type=jnp.float32)
    h += (1.0 / L_T) * jnp.dot(tr[...], w1[64:128],
                               preferred_element_type=jnp.float32)
    h += (1.0 / L_D) * jnp.dot(dr[...], w1[128:192],
                               preferred_element_type=jnp.float32)
    h += jnp.dot(brr[...], w1[192:208], preferred_element_type=jnp.float32)
    h = jnp.maximum(h + b1r[...], 0.0)
    outr[...] = jnp.dot(h, w2r[...], preferred_element_type=jnp.float32) + b2r[...]


def _tc_mlp(id_emb, t_sum, d_sum, br_emb, W1, b1, W2, b2):
    blk = 512
    return pl.pallas_call(
        _mlp_body,
        grid=(B // blk,),
        in_specs=[
            pl.BlockSpec((blk, D_EMB), lambda i: (i, 0)),
            pl.BlockSpec((blk, D_EMB), lambda i: (i, 0)),
            pl.BlockSpec((blk, D_EMB), lambda i: (i, 0)),
            pl.BlockSpec((blk, D_BR), lambda i: (i, 0)),
            pl.BlockSpec((208, HIDDEN), lambda i: (0, 0)),
            pl.BlockSpec((1, HIDDEN), lambda i: (0, 0)),
            pl.BlockSpec((HIDDEN, OUT), lambda i: (0, 0)),
            pl.BlockSpec((1, OUT), lambda i: (0, 0)),
        ],
        out_specs=pl.BlockSpec((blk, OUT), lambda i: (i, 0)),
        out_shape=jax.ShapeDtypeStruct((B, OUT), jnp.float32),
    )(id_emb, t_sum, d_sum, br_emb, W1, b1.reshape(1, HIDDEN), W2,
      b2.reshape(1, OUT))


def kernel(product_id, product_title, product_description, product_brand,
           id_table, title_table, desc_table, brand_table, W1, b1, W2, b2):
    id_idx = product_id.astype(jnp.int32)
    br_idx = product_brand.astype(jnp.int32)
    t_idx = product_title.astype(jnp.int32)
    d_idx = product_description.astype(jnp.int32)
    id_emb, t_sum, d_sum, br_emb = _sc_embed(
        id_idx, br_idx, t_idx, d_idx,
        id_table, brand_table, title_table, desc_table)
    return _tc_mlp(id_emb, t_sum, d_sum, br_emb, W1, b1, W2, b2)


# speedup vs baseline: 5.5083x; 1.5231x over previous
import functools

import jax
import jax.numpy as jnp
from jax import lax
from jax.experimental import pallas as pl
from jax.experimental.pallas import tpu as pltpu
from jax.experimental.pallas import tpu_sc as plsc

B = 4096
L_T = 50
L_D = 200
D_EMB = 64
D_BR = 16
HIDDEN = 128
OUT = 64

NC = 2
NS = 16
NW = NC * NS
IPW = B // NW

_MESH = dict(core_axis_name="c", subcore_axis_name="s")
_LINEAR = pltpu.CompilerParams(use_tc_tiling_on_sc=False)


def _row_sum2(rows_ref, n_rows):
    def body(r, accs):
        out = []
        for j in range(4):
            sl = pl.ds(j * 16, 16)
            out.append(accs[j] + (rows_ref[2 * r, sl] + rows_ref[2 * r + 1, sl]))
        return tuple(out)
    init = tuple(jnp.zeros((16,), jnp.float32) for _ in range(4))
    return lax.fori_loop(0, n_rows // 2, body, init)


def _sc_bags(t_idx, d_idx, t_tab, d_tab):
    @functools.partial(
        pl.kernel,
        compiler_params=_LINEAR,
        out_type=(
            jax.ShapeDtypeStruct((B, D_EMB), jnp.float32),
            jax.ShapeDtypeStruct((B, D_EMB), jnp.float32),
        ),
        mesh=plsc.VectorSubcoreMesh(**_MESH),
        scratch_types=[
            pltpu.VMEM((IPW, L_T), jnp.int32),
            pltpu.VMEM((IPW, L_D), jnp.int32),
            pltpu.VMEM((L_T, D_EMB), jnp.float32),
            pltpu.VMEM((L_T, D_EMB), jnp.float32),
            pltpu.VMEM((L_D, D_EMB), jnp.float32),
            pltpu.VMEM((L_D, D_EMB), jnp.float32),
            pltpu.VMEM((IPW, D_EMB), jnp.float32),
            pltpu.VMEM((IPW, D_EMB), jnp.float32),
            pltpu.SemaphoreType.DMA,
            pltpu.SemaphoreType.DMA,
            pltpu.SemaphoreType.DMA,
            pltpu.SemaphoreType.DMA,
        ],
    )
    def k(t_idx_hbm, d_idx_hbm, t_tab_hbm, d_tab_hbm,
          t_out, d_out,
          tiv, div, tbuf0, tbuf1, dbuf0, dbuf1, tacc, dacc,
          ts0, ts1, ds0, ds1):
        wid = lax.axis_index("s") * NC + lax.axis_index("c")
        base = wid * IPW

        pltpu.sync_copy(t_idx_hbm.at[pl.ds(base, IPW)], tiv)
        pltpu.sync_copy(d_idx_hbm.at[pl.ds(base, IPW)], div)

        def start(b, tbuf, dbuf, tsem, dsem):
            pltpu.async_copy(t_tab_hbm.at[tiv.at[b]], tbuf, tsem)
            pltpu.async_copy(d_tab_hbm.at[div.at[b, pl.ds(0, 128)]],
                             dbuf.at[pl.ds(0, 128)], dsem)
            pltpu.async_copy(d_tab_hbm.at[div.at[b, pl.ds(128, L_D - 128)]],
                             dbuf.at[pl.ds(128, L_D - 128)], dsem)

        def wait(tbuf, dbuf, tsem, dsem):
            pltpu.make_async_copy(t_tab_hbm.at[pl.ds(0, L_T)], tbuf, tsem).wait()
            pltpu.make_async_copy(d_tab_hbm.at[pl.ds(0, L_D)], dbuf, dsem).wait()

        def reduce(b, tbuf, dbuf):
            taccs = _row_sum2(tbuf, L_T)
            daccs = _row_sum2(dbuf, L_D)
            for j in range(4):
                tacc[b, pl.ds(j * 16, 16)] = taccs[j]
                dacc[b, pl.ds(j * 16, 16)] = daccs[j]

        start(0, tbuf0, dbuf0, ts0, ds0)

        @pl.loop(0, IPW, step=2)
        def _(b):
            start(b + 1, tbuf1, dbuf1, ts1, ds1)
            wait(tbuf0, dbuf0, ts0, ds0)
            reduce(b, tbuf0, dbuf0)

            @pl.when(b + 2 < IPW)
            def _():
                start(b + 2, tbuf0, dbuf0, ts0, ds0)

            wait(tbuf1, dbuf1, ts1, ds1)
            reduce(b + 1, tbuf1, dbuf1)

        pltpu.sync_copy(tacc, t_out.at[pl.ds(base, IPW)])
        pltpu.sync_copy(dacc, d_out.at[pl.ds(base, IPW)])

    return k(t_idx, d_idx, t_tab, d_tab)


def _sc_idbrand(id_idx, br_idx, id_tab, br_tab):
    @functools.partial(
        pl.kernel,
        compiler_params=_LINEAR,
        out_type=(
            jax.ShapeDtypeStruct((B, D_EMB), jnp.float32),
            jax.ShapeDtypeStruct((B, D_BR), jnp.float32),
        ),
        mesh=plsc.VectorSubcoreMesh(**_MESH),
        scratch_types=[
            pltpu.VMEM((IPW,), jnp.int32),
            pltpu.VMEM((IPW,), jnp.int32),
            pltpu.VMEM((IPW, D_EMB), jnp.float32),
            pltpu.VMEM((IPW, D_BR), jnp.float32),
            pltpu.SemaphoreType.DMA,
        ],
    )
    def k(id_idx_hbm, br_idx_hbm, id_tab_hbm, br_tab_hbm,
          id_out, br_out, idv, brv, idrows, brrows, sem):
        wid = lax.axis_index("s") * NC + lax.axis_index("c")
        base = wid * IPW
        pltpu.sync_copy(id_idx_hbm.at[pl.ds(base, IPW)], idv)
        pltpu.sync_copy(br_idx_hbm.at[pl.ds(base, IPW)], brv)
        pltpu.async_copy(id_tab_hbm.at[idv], idrows, sem)
        pltpu.sync_copy(br_tab_hbm.at[brv], brrows)
        pltpu.make_async_copy(id_tab_hbm.at[pl.ds(0, IPW)], idrows, sem).wait()
        pltpu.sync_copy(idrows, id_out.at[pl.ds(base, IPW)])
        pltpu.sync_copy(brrows, br_out.at[pl.ds(base, IPW)])

    return k(id_idx, br_idx, id_tab, br_tab)


def _mlp_body(idr, tr, dr, brr, w1r, b1r, w2r, b2r, outr):
    w1 = w1r[...]
    h = jnp.dot(idr[...], w1[0:64], preferred_element_type=jnp.float32)
    h += (1.0 / L_T) * jnp.dot(tr[...], w1[64:128],
                               preferred_element_type=jnp.float32)
    h += (1.0 / L_D) * jnp.dot(dr[...], w1[128:192],
                               preferred_element_type=jnp.float32)
    h += jnp.dot(brr[...], w1[192:208], preferred_element_type=jnp.float32)
    h = jnp.maximum(h + b1r[...], 0.0)
    outr[...] = jnp.dot(h, w2r[...], preferred_element_type=jnp.float32) + b2r[...]


def _tc_mlp(id_emb, t_sum, d_sum, br_emb, W1, b1, W2, b2):
    blk = 512
    return pl.pallas_call(
        _mlp_body,
        grid=(B // blk,),
        in_specs=[
            pl.BlockSpec((blk, D_EMB), lambda i: (i, 0)),
            pl.BlockSpec((blk, D_EMB), lambda i: (i, 0)),
            pl.BlockSpec((blk, D_EMB), lambda i: (i, 0)),
            pl.BlockSpec((blk, D_BR), lambda i: (i, 0)),
            pl.BlockSpec((208, HIDDEN), lambda i: (0, 0)),
            pl.BlockSpec((1, HIDDEN), lambda i: (0, 0)),
            pl.BlockSpec((HIDDEN, OUT), lambda i: (0, 0)),
            pl.BlockSpec((1, OUT), lambda i: (0, 0)),
        ],
        out_specs=pl.BlockSpec((blk, OUT), lambda i: (i, 0)),
        out_shape=jax.ShapeDtypeStruct((B, OUT), jnp.float32),
    )(id_emb, t_sum, d_sum, br_emb, W1, b1.reshape(1, HIDDEN), W2,
      b2.reshape(1, OUT))


def kernel(product_id, product_title, product_description, product_brand,
           id_table, title_table, desc_table, brand_table, W1, b1, W2, b2):
    t_sum, d_sum = _sc_bags(product_title.astype(jnp.int32),
                            product_description.astype(jnp.int32),
                            title_table, desc_table)
    id_emb, br_emb = _sc_idbrand(product_id.astype(jnp.int32),
                                 product_brand.astype(jnp.int32),
                                 id_table, brand_table)
    return _tc_mlp(id_emb, t_sum, d_sum, br_emb, W1, b1, W2, b2)


# speedup vs baseline: 8.3330x; 1.5128x over previous
import functools

import jax
import jax.numpy as jnp
from jax import lax
from jax.experimental import pallas as pl
from jax.experimental.pallas import tpu as pltpu
from jax.experimental.pallas import tpu_sc as plsc

B = 4096
L_T = 50
L_D = 200
D_EMB = 64
D_BR = 16
HIDDEN = 128
OUT = 64

NC = 2
NS = 16
NW = NC * NS
IPW = B // NW

_MESH = dict(core_axis_name="c", subcore_axis_name="s")
_LINEAR = pltpu.CompilerParams(use_tc_tiling_on_sc=False)


def _row_sum2(rows_ref, n_rows):
    def body(r, accs):
        out = []
        for j in range(4):
            sl = pl.ds(j * 16, 16)
            out.append(accs[j] + (rows_ref[2 * r, sl] + rows_ref[2 * r + 1, sl]))
        return tuple(out)
    init = tuple(jnp.zeros((16,), jnp.float32) for _ in range(4))
    return lax.fori_loop(0, n_rows // 2, body, init)


def _sc_bags(t_idx, d_idx, t_tab, d_tab):
    @functools.partial(
        pl.kernel,
        compiler_params=_LINEAR,
        out_type=(
            jax.ShapeDtypeStruct((B, D_EMB), jnp.float32),
            jax.ShapeDtypeStruct((B, D_EMB), jnp.float32),
        ),
        mesh=plsc.VectorSubcoreMesh(**_MESH),
        scratch_types=[
            pltpu.VMEM((IPW, L_T), jnp.int32),
            pltpu.VMEM((IPW, L_D), jnp.int32),
            pltpu.VMEM((L_T, D_EMB), jnp.float32),
            pltpu.VMEM((L_T, D_EMB), jnp.float32),
            pltpu.VMEM((L_D, D_EMB), jnp.float32),
            pltpu.VMEM((L_D, D_EMB), jnp.float32),
            pltpu.VMEM((IPW, D_EMB), jnp.float32),
            pltpu.VMEM((IPW, D_EMB), jnp.float32),
            pltpu.SemaphoreType.DMA,
            pltpu.SemaphoreType.DMA,
            pltpu.SemaphoreType.DMA,
            pltpu.SemaphoreType.DMA,
        ],
    )
    def k(t_idx_hbm, d_idx_hbm, t_tab_hbm, d_tab_hbm,
          t_out, d_out,
          tiv, div, tbuf0, tbuf1, dbuf0, dbuf1, tacc, dacc,
          ts0, ts1, ds0, ds1):
        wid = lax.axis_index("s") * NC + lax.axis_index("c")
        base = wid * IPW

        pltpu.sync_copy(t_idx_hbm.at[pl.ds(base, IPW)], tiv)
        pltpu.sync_copy(d_idx_hbm.at[pl.ds(base, IPW)], div)

        def start(b, tbuf, dbuf, tsem, dsem):
            pltpu.async_copy(t_tab_hbm.at[tiv.at[b]], tbuf, tsem)
            pltpu.async_copy(d_tab_hbm.at[div.at[b, pl.ds(0, 128)]],
                             dbuf.at[pl.ds(0, 128)], dsem)
            pltpu.async_copy(d_tab_hbm.at[div.at[b, pl.ds(128, L_D - 128)]],
                             dbuf.at[pl.ds(128, L_D - 128)], dsem)

        def wait(tbuf, dbuf, tsem, dsem):
            pltpu.make_async_copy(t_tab_hbm.at[pl.ds(0, L_T)], tbuf, tsem).wait()
            pltpu.make_async_copy(d_tab_hbm.at[pl.ds(0, L_D)], dbuf, dsem).wait()

        def reduce(b, tbuf, dbuf):
            taccs = _row_sum2(tbuf, L_T)
            daccs = _row_sum2(dbuf, L_D)
            for j in range(4):
                tacc[b, pl.ds(j * 16, 16)] = taccs[j]
                dacc[b, pl.ds(j * 16, 16)] = daccs[j]

        start(0, tbuf0, dbuf0, ts0, ds0)

        @pl.loop(0, IPW, step=2)
        def _(b):
            start(b + 1, tbuf1, dbuf1, ts1, ds1)
            wait(tbuf0, dbuf0, ts0, ds0)
            reduce(b, tbuf0, dbuf0)

            @pl.when(b + 2 < IPW)
            def _():
                start(b + 2, tbuf0, dbuf0, ts0, ds0)

            wait(tbuf1, dbuf1, ts1, ds1)
            reduce(b + 1, tbuf1, dbuf1)

        pltpu.sync_copy(tacc, t_out.at[pl.ds(base, IPW)])
        pltpu.sync_copy(dacc, d_out.at[pl.ds(base, IPW)])

    return k(t_idx, d_idx, t_tab, d_tab)


def _sc_idbrand(id_idx, br_idx, id_tab2, br_tab8):
    @functools.partial(
        pl.kernel,
        compiler_params=pltpu.CompilerParams(use_tc_tiling_on_sc=True),
        out_type=(
            jax.ShapeDtypeStruct((B, 128), jnp.float32),
            jax.ShapeDtypeStruct((B, 128), jnp.float32),
        ),
        mesh=plsc.VectorSubcoreMesh(**_MESH),
        scratch_types=[
            pltpu.VMEM((IPW,), jnp.int32),
            pltpu.VMEM((IPW,), jnp.int32),
            pltpu.VMEM((IPW, 128), jnp.float32),
            pltpu.VMEM((IPW, 128), jnp.float32),
            pltpu.SemaphoreType.DMA,
            pltpu.SemaphoreType.DMA,
        ],
    )
    def k(id_idx_hbm, br_idx_hbm, id_tab_hbm, br_tab_hbm,
          id_out, br_out, idv, brv, idrows, brrows, sem1, sem2):
        wid = lax.axis_index("s") * NC + lax.axis_index("c")
        base = wid * IPW
        pltpu.sync_copy(id_idx_hbm.at[pl.ds(base, IPW)], idv)
        pltpu.sync_copy(br_idx_hbm.at[pl.ds(base, IPW)], brv)
        @pl.loop(0, IPW, step=16)
        def _(i):
            sl = pl.ds(i, 16)
            brv[sl] = jax.lax.shift_right_logical(brv[sl], 3)
        pltpu.async_copy(id_tab_hbm.at[idv], idrows, sem1)
        pltpu.async_copy(br_tab_hbm.at[brv], brrows, sem2)
        pltpu.make_async_copy(id_tab_hbm.at[pl.ds(0, IPW)], idrows, sem1).wait()
        pltpu.make_async_copy(br_tab_hbm.at[pl.ds(0, IPW)], brrows, sem2).wait()
        pltpu.sync_copy(idrows, id_out.at[pl.ds(base, IPW)])
        pltpu.sync_copy(brrows, br_out.at[pl.ds(base, IPW)])

    return k(id_idx, br_idx, id_tab2, br_tab8)


def _widen_body(tr, outr):
    y = jnp.transpose(tr[...])
    outr[:, 0:64] = y
    outr[:, 64:128] = jnp.zeros_like(y)


def _tc_widen(table):
    v, d = table.shape
    assert d == 64
    t_t = table.T
    ch = 8192
    return pl.pallas_call(
        _widen_body,
        grid=(pl.cdiv(v, ch),),
        in_specs=[pl.BlockSpec((64, ch), lambda i: (0, i))],
        out_specs=pl.BlockSpec((ch, 128), lambda i: (i, 0)),
        out_shape=jax.ShapeDtypeStruct((v, 128), jnp.float32),
    )(t_t)


def _mlp_body(pbrr, idr, tr, dr, brr, w1r, b1r, w2r, b2r, outr):
    w1 = w1r[...]
    id_emb = idr[...][:, 0:D_EMB]
    brp = brr[...]
    bmod = pbrr[...] & 7
    br_emb = jnp.zeros_like(brp[:, 0:D_BR])
    for kk in range(8):
        br_emb = jnp.where(bmod == kk, brp[:, kk * D_BR:(kk + 1) * D_BR], br_emb)
    h = jnp.dot(id_emb, w1[0:64], preferred_element_type=jnp.float32)
    h += (1.0 / L_T) * jnp.dot(tr[...], w1[64:128],
                               preferred_element_type=jnp.float32)
    h += (1.0 / L_D) * jnp.dot(dr[...], w1[128:192],
                               preferred_element_type=jnp.float32)
    h += jnp.dot(br_emb, w1[192:208], preferred_element_type=jnp.float32)
    h = jnp.maximum(h + b1r[...], 0.0)
    outr[...] = jnp.dot(h, w2r[...], preferred_element_type=jnp.float32) + b2r[...]


def _tc_mlp(pbr, id_emb, t_sum, d_sum, br_emb, W1, b1, W2, b2):
    blk = 512
    return pl.pallas_call(
        _mlp_body,
        grid=(B // blk,),
        in_specs=[
            pl.BlockSpec((blk, 1), lambda i: (i, 0)),
            pl.BlockSpec((blk, 128), lambda i: (i, 0)),
            pl.BlockSpec((blk, D_EMB), lambda i: (i, 0)),
            pl.BlockSpec((blk, D_EMB), lambda i: (i, 0)),
            pl.BlockSpec((blk, 128), lambda i: (i, 0)),
            pl.BlockSpec((208, HIDDEN), lambda i: (0, 0)),
            pl.BlockSpec((1, HIDDEN), lambda i: (0, 0)),
            pl.BlockSpec((HIDDEN, OUT), lambda i: (0, 0)),
            pl.BlockSpec((1, OUT), lambda i: (0, 0)),
        ],
        out_specs=pl.BlockSpec((blk, OUT), lambda i: (i, 0)),
        out_shape=jax.ShapeDtypeStruct((B, OUT), jnp.float32),
    )(pbr.reshape(B, 1), id_emb, t_sum, d_sum, br_emb,
      W1, b1.reshape(1, HIDDEN), W2, b2.reshape(1, OUT))


def kernel(product_id, product_title, product_description, product_brand,
           id_table, title_table, desc_table, brand_table, W1, b1, W2, b2):
    t_sum, d_sum = _sc_bags(product_title.astype(jnp.int32),
                            product_description.astype(jnp.int32),
                            title_table, desc_table)
    pid = product_id.astype(jnp.int32)
    pbr = product_brand.astype(jnp.int32)
    id_emb, br_emb = _sc_idbrand(pid, pbr,
                                 _tc_widen(id_table),
                                 brand_table.reshape(-1, 128))
    return _tc_mlp(pbr, id_emb, t_sum, d_sum, br_emb, W1, b1, W2, b2)
